# baseline (speedup 1.0000x reference)
import jax
import jax.numpy as jnp
from jax.experimental import pallas as pl
from jax.experimental.pallas import tpu as pltpu

N = 4096
D = 128
BC = 512
NR = 1
HR = N // NR


def _gcn_norm_kernel(x_ref, adj_ref, w_ref, out_ref, h_ref):
    i = pl.program_id(0)
    j = pl.program_id(1)

    @pl.when((i == 0) & (j == 0))
    def _compute_h():
        h_ref[...] = jnp.dot(
            x_ref[...], w_ref[...], preferred_element_type=jnp.float32
        )

    part = jax.lax.dot_general(
        adj_ref[...], h_ref[pl.ds(j * HR, HR), :],
        dimension_numbers=(((0,), (0,)), ((), ())),
        preferred_element_type=jnp.float32,
    )

    @pl.when(j == 0)
    def _assign():
        out_ref[pl.ds(i * BC, BC), :] = part

    @pl.when(j > 0)
    def _accum():
        out_ref[pl.ds(i * BC, BC), :] += part

    @pl.when((i == pl.num_programs(0) - 1) & (j == pl.num_programs(1) - 1))
    def _finalize():
        a = out_ref[...]
        c = a - jnp.mean(a, axis=0, keepdims=True)
        rnorm = jnp.sqrt(1e-6 + jnp.sum(c * c, axis=1, keepdims=True))
        out_ref[...] = jnp.maximum(c / rnorm, 0.0)


def kernel(x, adj, W, b):
    del b
    out = pl.pallas_call(
        _gcn_norm_kernel,
        grid=(N // BC, NR),
        in_specs=[
            pl.BlockSpec((N, D), lambda i, j: (0, 0)),
            pl.BlockSpec((HR, BC), lambda i, j: (j, i)),
            pl.BlockSpec((D, D), lambda i, j: (0, 0)),
        ],
        out_specs=pl.BlockSpec((N, D), lambda i, j: (0, 0)),
        out_shape=jax.ShapeDtypeStruct((N, D), jnp.float32),
        scratch_shapes=[pltpu.VMEM((N, D), jnp.float32)],
    )(x, adj, W)
    return (out, adj)


# speedup vs baseline: 1.0345x; 1.0345x over previous
import jax
import jax.numpy as jnp
from jax.experimental import pallas as pl
from jax.experimental.pallas import tpu as pltpu

N = 4096
D = 128
BC = 512


def _gcn_norm_kernel(x_ref, adj_ref, w_ref, out_ref, h_ref, m_ref):
    i = pl.program_id(0)

    @pl.when(i == 0)
    def _init():
        h_ref[...] = jnp.dot(
            x_ref[...], w_ref[...], preferred_element_type=jnp.float32
        ).astype(jnp.bfloat16)
        m_ref[...] = jnp.zeros_like(m_ref)

    part = jax.lax.dot_general(
        adj_ref[...].astype(jnp.bfloat16), h_ref[...],
        dimension_numbers=(((0,), (0,)), ((), ())),
        preferred_element_type=jnp.float32,
    )
    out_ref[pl.ds(i * BC, BC), :] = part
    m_ref[0:1, :] += jnp.sum(part, axis=0, keepdims=True)

    @pl.when(i == pl.num_programs(0) - 1)
    def _finalize():
        mean = m_ref[0:1, :] * (1.0 / N)
        c = out_ref[...] - mean
        rnorm = jnp.sqrt(1e-6 + jnp.sum(c * c, axis=1, keepdims=True))
        out_ref[...] = jnp.maximum(c / rnorm, 0.0)


def kernel(x, adj, W, b):
    del b
    out = pl.pallas_call(
        _gcn_norm_kernel,
        grid=(N // BC,),
        in_specs=[
            pl.BlockSpec((N, D), lambda i: (0, 0)),
            pl.BlockSpec((N, BC), lambda i: (0, i)),
            pl.BlockSpec((D, D), lambda i: (0, 0)),
        ],
        out_specs=pl.BlockSpec((N, D), lambda i: (0, 0)),
        out_shape=jax.ShapeDtypeStruct((N, D), jnp.float32),
        scratch_shapes=[
            pltpu.VMEM((N, D), jnp.bfloat16),
            pltpu.VMEM((8, D), jnp.float32),
        ],
    )(x, adj, W)
    return (out, adj)
